# baseline (speedup 1.0000x reference)
import functools
from math import pi as PI

import jax
import jax.numpy as jnp
from jax import lax
from jax.experimental import pallas as pl
from jax.experimental.pallas import tpu as pltpu
from jax.experimental.pallas import tpu_sc as plsc

CUTOFF = 10.0
LOG2 = 0.6931471805599453

N = 10000
E = 160000
H = 256
G = 50
GP = 128
FNUM = 256
NC = 4
HH = H // 2

BN = 1000

NWK = 32
CE = E // NWK
NGR = 313
CEP = NGR * 16
TP = 128
BE2 = 640
EPAD = 179200
NB2 = EPAD // BE2
EPAD2 = EPAD + TP
CEB2 = 5120
NBCH = CEB2 // TP
R_PST = NWK * NC
R_PEN = 2 * NWK * NC
R_GS = 3 * NWK * NC
R_GE = 3 * NWK * NC + NC
NAUX = 3 * NWK * NC + 2 * NC
GAP3G = 40


def _ssp(v):
    return jax.nn.softplus(v) - LOG2


def _histbases_body(c_ref, aux_ref, offs_ref, mb_ref):
    cols = c_ref[...]
    lane = jax.lax.broadcasted_iota(jnp.int32, (1, 16), 1)
    cnt = jnp.zeros((NWK, 16), jnp.int32)
    for c in range(NC):
        cc = jnp.sum((cols[:, 0, :] == c).astype(jnp.int32), axis=1)
        cnt = cnt + jnp.where(lane == c, cc[:, None], 0)
    pcnt = jnp.bitwise_and(cnt + (TP - 1), -TP) * (lane < NC)
    ri = jax.lax.broadcasted_iota(jnp.int32, (NWK, NWK), 0)
    ci = jax.lax.broadcasted_iota(jnp.int32, (NWK, NWK), 1)
    ltri = (ci < ri).astype(jnp.float32)
    rel = jnp.dot(ltri, pcnt.astype(jnp.float32),
                  preferred_element_type=jnp.float32).astype(jnp.int32)
    totv = jnp.sum(pcnt, axis=0)
    startvec = jnp.zeros((1, 16), jnp.int32)
    sacc = jnp.int32(0)
    starts = []
    gss = []
    ges = []
    for c in range(NC):
        starts.append(sacc)
        tc = jnp.sum(totv * (lane[0] == c).astype(jnp.int32))
        startvec = startvec + jnp.where(lane == c, sacc, 0)
        gss.append(sacc + tc)
        sacc = ((sacc + tc + (BE2 - 1)) // BE2) * BE2
        ges.append(jnp.int32(EPAD) if c == NC - 1 else sacc)
    mb = jnp.stack([starts[c] + rel[:, c] for c in range(NC)], axis=1)
    cntm = jnp.stack([cnt[:, c] for c in range(NC)], axis=1)
    pcm = jnp.stack([pcnt[:, c] for c in range(NC)], axis=1)
    base_rows = jnp.broadcast_to(mb[:, :, None], (NWK, NC, 16)).reshape(NWK * NC, 16)
    pst_rows = jnp.broadcast_to((mb + cntm)[:, :, None], (NWK, NC, 16)).reshape(NWK * NC, 16)
    pen_rows = jnp.broadcast_to((mb + pcm)[:, :, None], (NWK, NC, 16)).reshape(NWK * NC, 16)
    gs_rows = jnp.broadcast_to(jnp.stack(gss)[:, None], (NC, 16))
    ge_rows = jnp.broadcast_to(jnp.stack(ges)[:, None], (NC, 16))
    allrows = jnp.concatenate([base_rows, pst_rows, pen_rows, gs_rows, ge_rows], axis=0)
    aux_ref[...] = allrows[:, None, :]
    offs_ref[...] = startvec[None]
    mb_ref[...] = mb[:, None, :]


def _histbases(colors2d):
    return pl.pallas_call(
        _histbases_body,
        grid=(1,),
        in_specs=[pl.BlockSpec((NWK, 1, CEP), lambda i: (0, 0, 0))],
        out_specs=[
            pl.BlockSpec((NAUX, 1, 16), lambda i: (0, 0, 0)),
            pl.BlockSpec((1, 1, 16), lambda i: (0, 0, 0)),
            pl.BlockSpec((NWK, 1, NC), lambda i: (0, 0, 0)),
        ],
        out_shape=[
            jax.ShapeDtypeStruct((NAUX, 1, 16), jnp.int32),
            jax.ShapeDtypeStruct((1, 1, 16), jnp.int32),
            jax.ShapeDtypeStruct((NWK, 1, NC), jnp.int32),
        ],
    )(colors2d.reshape(NWK, 1, CEP))


def _ranks_body(c_ref, mb_ref, pos_ref):
    g16 = c_ref[0]
    li = jax.lax.broadcasted_iota(jnp.int32, (16, 16), 0)
    lj = jax.lax.broadcasted_iota(jnp.int32, (16, 16), 1)
    tri_incl = (lj <= li).astype(jnp.float32).T
    gi = jax.lax.broadcasted_iota(jnp.int32, (NGR, NGR), 0)
    gj = jax.lax.broadcasted_iota(jnp.int32, (NGR, NGR), 1)
    tri_ex = (gj < gi).astype(jnp.float32)
    pos = jnp.full((NGR, 16), EPAD, jnp.int32)
    pos = pos + jax.lax.broadcasted_iota(jnp.int32, (NGR, 16), 1)
    for c in range(NC):
        oh = (g16 == c).astype(jnp.float32)
        pref = jnp.dot(oh, tri_incl, preferred_element_type=jnp.float32)
        gcnt = jnp.sum(oh, axis=1, keepdims=True)
        rel = jnp.dot(tri_ex, gcnt, preferred_element_type=jnp.float32)
        base = mb_ref[0, 0, c]
        cand = base + (rel + pref - 1.0).astype(jnp.int32)
        pos = jnp.where(g16 == c, cand, pos)
    pos_ref[0] = pos


def _ranks(colors2d, mbase):
    return pl.pallas_call(
        _ranks_body,
        grid=(NWK,),
        in_specs=[
            pl.BlockSpec((1, NGR, 16), lambda i: (i, 0, 0)),
            pl.BlockSpec((1, 1, NC), lambda i: (i, 0, 0)),
        ],
        out_specs=pl.BlockSpec((1, NGR, 16), lambda i: (i, 0, 0)),
        out_shape=jax.ShapeDtypeStruct((NWK, NGR, 16), jnp.int32),
    )(colors2d.reshape(NWK, NGR, 16), mbase)


def _scatter_body(pos_hbm, src_hbm, dst_hbm, ew_hbm, eid_hbm, aux_hbm,
                  srcp, dstp, ewp, validp, eidp,
                  srcv, dstv, ewv, eidv, posbuf, ones128,
                  istage, fstage, auxbuf, sem):
    cid = lax.axis_index("c")
    sid = lax.axis_index("s")
    wid = cid * 16 + sid
    iota = lax.iota(jnp.int32, 16)
    zi = jnp.zeros((16,), jnp.int32)
    zf = jnp.zeros((16,), jnp.float32)
    trash = iota + EPAD

    for w in range(TP // 16):
        istage[pl.ds(w * 16, 16)] = zi
        fstage[pl.ds(w * 16, 16)] = zf
        ones128[pl.ds(w * 16, 16)] = zf + 1.0

    pltpu.sync_copy(src_hbm.at[pl.ds(wid * CE, CE)], srcv.at[pl.ds(0, CE)])
    pltpu.sync_copy(dst_hbm.at[pl.ds(wid * CE, CE)], dstv.at[pl.ds(0, CE)])
    pltpu.sync_copy(ew_hbm.at[pl.ds(wid * CE, CE)], ewv.at[pl.ds(0, CE)])
    pltpu.sync_copy(eid_hbm.at[pl.ds(wid * CE, CE)], eidv.at[pl.ds(0, CE)])

    def batch(b, c2):
        sl = pl.ds(b * TP, TP)
        pltpu.sync_copy(pos_hbm.at[pl.ds(wid * CEB2 + b * TP, TP)],
                        posbuf.at[b])
        idx = posbuf.at[b]
        cp1 = pltpu.async_copy(eidv.at[sl], eidp.at[idx], sem)
        cp2 = pltpu.async_copy(srcv.at[sl], srcp.at[idx], sem)
        cp3 = pltpu.async_copy(dstv.at[sl], dstp.at[idx], sem)
        cp4 = pltpu.async_copy(ewv.at[sl], ewp.at[idx], sem)
        cp5 = pltpu.async_copy(ones128, validp.at[idx], sem)
        cp1.wait()
        cp2.wait()
        cp3.wait()
        cp4.wait()
        cp5.wait()
        return c2
    lax.fori_loop(0, NBCH, batch, 0)

    pltpu.sync_copy(aux_hbm.at[pl.ds((R_PST + wid * NC) * 16, NC * 16)],
                    auxbuf.at[pl.ds(0, NC * 16)])
    pltpu.sync_copy(aux_hbm.at[pl.ds((R_PEN + wid * NC) * 16, NC * 16)],
                    auxbuf.at[pl.ds(NC * 16, NC * 16)])
    z16i = istage.at[pl.ds(0, 16)]
    z16f = fstage.at[pl.ds(0, 16)]
    for c in range(NC):
        pstart = auxbuf[pl.ds(c * 16, 16)]
        pend = auxbuf[pl.ds((NC + c) * 16, 16)]
        for k in range(TP // 16):
            pos = pstart + (k * 16 + iota)
            pos = jnp.where(pos < pend, pos, trash)
            cp1 = pltpu.async_copy(z16i, eidp.at[pos], sem)
            cp2 = pltpu.async_copy(z16i, srcp.at[pos], sem)
            cp3 = pltpu.async_copy(z16i, dstp.at[pos], sem)
            cp4 = pltpu.async_copy(z16f, ewp.at[pos], sem)
            cp5 = pltpu.async_copy(z16f, validp.at[pos], sem)
            cp1.wait()
            cp2.wait()
            cp3.wait()
            cp4.wait()
            cp5.wait()

    pltpu.sync_copy(aux_hbm.at[pl.ds(R_GS * 16, 2 * NC * 16)],
                    auxbuf.at[pl.ds(0, 2 * NC * 16)])
    for c in range(NC):
        gs = auxbuf[pl.ds(c * 16, 16)]
        ge = auxbuf[pl.ds((NC + c) * 16, 16)]
        ngroups = 3 if c < NC - 1 else GAP3G

        def gap(k, c2, gs=gs, ge=ge, ngroups=ngroups):
            pos = gs + (wid * (ngroups * 16) + k * 16 + iota)
            pos = jnp.where(pos < ge, pos, trash)
            cp1 = pltpu.async_copy(z16i, eidp.at[pos], sem)
            cp2 = pltpu.async_copy(z16i, srcp.at[pos], sem)
            cp3 = pltpu.async_copy(z16i, dstp.at[pos], sem)
            cp4 = pltpu.async_copy(z16f, ewp.at[pos], sem)
            cp5 = pltpu.async_copy(z16f, validp.at[pos], sem)
            cp1.wait()
            cp2.wait()
            cp3.wait()
            cp4.wait()
            cp5.wait()
            return c2
        lax.fori_loop(0, ngroups, gap, 0)


def _scatter(posE, src, dst, ew, eids, aux):
    mesh = plsc.VectorSubcoreMesh(core_axis_name="c", subcore_axis_name="s",
                                  num_cores=2, num_subcores=16)
    f = pl.kernel(
        _scatter_body,
        name="sc_scatter",
        out_type=[
            jax.ShapeDtypeStruct((EPAD2,), jnp.int32),
            jax.ShapeDtypeStruct((EPAD2,), jnp.int32),
            jax.ShapeDtypeStruct((EPAD2,), jnp.float32),
            jax.ShapeDtypeStruct((EPAD2,), jnp.float32),
            jax.ShapeDtypeStruct((EPAD2,), jnp.int32),
        ],
        mesh=mesh,
        scratch_types=[
            pltpu.VMEM((CEB2,), jnp.int32),
            pltpu.VMEM((CEB2,), jnp.int32),
            pltpu.VMEM((CEB2,), jnp.float32),
            pltpu.VMEM((CEB2,), jnp.int32),
            pltpu.VMEM((NBCH, TP), jnp.int32),
            pltpu.VMEM((TP,), jnp.float32),
            pltpu.VMEM((TP,), jnp.int32),
            pltpu.VMEM((TP,), jnp.float32),
            pltpu.VMEM((2 * NC * 16,), jnp.int32),
            pltpu.SemaphoreType.DMA,
        ],
    )
    return f(posE, src, dst, ew, eids, aux)


EAG_B = EPAD // NWK
EAG_K = 112


def _eagather_body(ea_hbm, eid_hbm, eap_hbm, idx_v, rows_v, sem):
    cid = lax.axis_index("c")
    sid = lax.axis_index("s")
    wid = cid * 16 + sid

    def chunk(k, carry):
        off = (wid * (EAG_B // EAG_K) + k) * EAG_K
        pltpu.sync_copy(eid_hbm.at[pl.ds(off, EAG_K)], idx_v)
        pltpu.async_copy(ea_hbm.at[idx_v], rows_v, sem).wait()
        pltpu.sync_copy(rows_v, eap_hbm.at[pl.ds(off, EAG_K)])
        return carry

    lax.fori_loop(0, EAG_B // EAG_K, chunk, 0)


def _eagather(ea_pad, eidp):
    mesh = plsc.VectorSubcoreMesh(core_axis_name="c", subcore_axis_name="s",
                                  num_cores=2, num_subcores=16)
    f = pl.kernel(
        _eagather_body,
        name="sc_eagather",
        out_type=jax.ShapeDtypeStruct((EPAD, GP), jnp.float32),
        mesh=mesh,
        scratch_types=[
            pltpu.VMEM((EAG_K,), jnp.int32),
            pltpu.VMEM((EAG_K, GP), jnp.float32),
            pltpu.SemaphoreType.DMA,
        ],
    )
    return f(ea_pad, eidp)


def _block_color(i, offs_ref):
    s = i * BE2
    c = (s >= offs_ref[1]).astype(jnp.int32)
    c = c + (s >= offs_ref[2]).astype(jnp.int32)
    c = c + (s >= offs_ref[3]).astype(jnp.int32)
    return c


def _mlp_body(offs_ref, ea_ref, ew_ref, va_ref, w1_ref, b1_ref, w2_ref, b2_ref,
              o0_ref, o1_ref):
    ea = ea_ref[...]
    c_env = (0.5 * (jnp.cos(ew_ref[...] * (PI / CUTOFF)) + 1.0)) * va_ref[...]
    h1 = _ssp(jnp.dot(ea, w1_ref[0], preferred_element_type=jnp.float32)
              + b1_ref[0, 0])
    f = jnp.dot(h1, w2_ref[0], preferred_element_type=jnp.float32) + b2_ref[0, 0]
    f = f * c_env
    o0_ref[...] = f[:, :HH]
    o1_ref[...] = f[:, HH:]


def _mlp(offs, eap, ewp_col, valid_col, w1, b1, w2, b2):
    grid_spec = pltpu.PrefetchScalarGridSpec(
        num_scalar_prefetch=1,
        grid=(NB2,),
        in_specs=[
            pl.BlockSpec((BE2, GP), lambda i, offs: (i, 0)),
            pl.BlockSpec((BE2, 1), lambda i, offs: (i, 0)),
            pl.BlockSpec((BE2, 1), lambda i, offs: (i, 0)),
            pl.BlockSpec((1, GP, FNUM), lambda i, offs: (_block_color(i, offs), 0, 0)),
            pl.BlockSpec((1, 1, FNUM), lambda i, offs: (_block_color(i, offs), 0, 0)),
            pl.BlockSpec((1, FNUM, FNUM), lambda i, offs: (_block_color(i, offs), 0, 0)),
            pl.BlockSpec((1, 1, FNUM), lambda i, offs: (_block_color(i, offs), 0, 0)),
        ],
        out_specs=[
            pl.BlockSpec((BE2, HH), lambda i, offs: (i, 0)),
            pl.BlockSpec((BE2, HH), lambda i, offs: (i, 0)),
        ],
    )
    return pl.pallas_call(
        _mlp_body,
        grid_spec=grid_spec,
        out_shape=[
            jax.ShapeDtypeStruct((EPAD, HH), jnp.float32),
            jax.ShapeDtypeStruct((EPAD, HH), jnp.float32),
        ],
    )(offs, eap, ewp_col, valid_col, w1, b1, w2, b2)


def _lin1_body(x_ref, w_ref, o0_ref, o1_ref):
    h = jnp.dot(x_ref[...], w_ref[...], preferred_element_type=jnp.float32)
    o0_ref[...] = h[:, :HH]
    o1_ref[...] = h[:, HH:]


def _lin1(x, lin1_W):
    return pl.pallas_call(
        _lin1_body,
        grid=(N // BN,),
        in_specs=[
            pl.BlockSpec((BN, H), lambda i: (i, 0)),
            pl.BlockSpec((H, FNUM), lambda i: (0, 0)),
        ],
        out_specs=[
            pl.BlockSpec((BN, HH), lambda i: (i, 0)),
            pl.BlockSpec((BN, HH), lambda i: (i, 0)),
        ],
        out_shape=[
            jax.ShapeDtypeStruct((N, HH), jnp.float32),
            jax.ShapeDtypeStruct((N, HH), jnp.float32),
        ],
    )(x, lin1_W)


NUM_TILES = 16
EP = EPAD // NUM_TILES
KCH = 80
NPAD = 10240
NP = NPAD // NUM_TILES
SB = 128


def _msg_half(h_hbm, w_hbm, src_hbm, dst_hbm, agg_hbm,
              srcbuf, dstbuf, hrows, wrows, sbuf, acc, sem1, sem2, sid):
    base = sid * EP

    def chunk(k, carry):
        off = base + k * KCH
        pltpu.sync_copy(src_hbm.at[pl.ds(off, KCH)], srcbuf)
        pltpu.sync_copy(dst_hbm.at[pl.ds(off, KCH)], dstbuf)
        cp1 = pltpu.async_copy(h_hbm.at[srcbuf], hrows, sem1)
        cp2 = pltpu.async_copy(w_hbm.at[pl.ds(off, KCH)], wrows, sem2)
        cp1.wait()
        cp2.wait()

        def mrow(r, c2):
            for c8 in range(HH // 16):
                s = pl.ds(c8 * 16, 16)
                wrows[r, s] = wrows[r, s] * hrows[r, s]
            return c2

        lax.fori_loop(0, KCH, mrow, 0)
        pltpu.sync_copy(wrows, acc.at[dstbuf], add=True)
        return carry

    lax.fori_loop(0, EP // KCH, chunk, 0)
    plsc.subcore_barrier()
    for r in range(NP // SB):
        sl = pl.ds(sid * NP + r * SB, SB)
        cp = pltpu.async_copy(acc.at[sl], sbuf, sem1)
        cp.wait()
        pltpu.sync_copy(sbuf, agg_hbm.at[sl])


def _msg_kernel_body(h0, h1, w0, w1, src_hbm, dst_hbm, agg0, agg1,
                     srcbuf, dstbuf, hrows, wrows, sbuf, acc, sem1, sem2):
    cid = lax.axis_index("c")
    sid = lax.axis_index("s")
    z = jnp.zeros((16,), jnp.float32)

    def zrow(r, carry):
        for c8 in range(HH // 16):
            sbuf[r, pl.ds(c8 * 16, 16)] = z
        return carry

    lax.fori_loop(0, SB, zrow, 0)
    for r in range(NP // SB):
        pltpu.sync_copy(sbuf, acc.at[pl.ds(sid * NP + r * SB, SB)])
    plsc.subcore_barrier()

    @pl.when(cid == 0)
    def _():
        _msg_half(h0, w0, src_hbm, dst_hbm, agg0,
                  srcbuf, dstbuf, hrows, wrows, sbuf, acc, sem1, sem2, sid)

    @pl.when(cid == 1)
    def _():
        _msg_half(h1, w1, src_hbm, dst_hbm, agg1,
                  srcbuf, dstbuf, hrows, wrows, sbuf, acc, sem1, sem2, sid)


def _msg_agg(h0, h1, w0, w1, src, dst):
    mesh = plsc.VectorSubcoreMesh(core_axis_name="c", subcore_axis_name="s",
                                  num_cores=2, num_subcores=16)
    f = pl.kernel(
        _msg_kernel_body,
        name="sc_msg",
        out_type=[
            jax.ShapeDtypeStruct((NPAD, HH), jnp.float32),
            jax.ShapeDtypeStruct((NPAD, HH), jnp.float32),
        ],
        mesh=mesh,
        scratch_types=[
            pltpu.VMEM((KCH,), jnp.int32),
            pltpu.VMEM((KCH,), jnp.int32),
            pltpu.VMEM((KCH, HH), jnp.float32),
            pltpu.VMEM((KCH, HH), jnp.float32),
            pltpu.VMEM((SB, HH), jnp.float32),
            pltpu.VMEM_SHARED((NPAD, HH), jnp.float32),
            pltpu.SemaphoreType.DMA,
            pltpu.SemaphoreType.DMA,
        ],
    )
    return f(h0, h1, w0, w1, src, dst)


def _tail_body(a0_ref, a1_ref, w2a_ref, w2b_ref, b2_ref, w_ref, b_ref, o_ref):
    t = (jnp.dot(a0_ref[...], w2a_ref[...], preferred_element_type=jnp.float32)
         + jnp.dot(a1_ref[...], w2b_ref[...], preferred_element_type=jnp.float32)
         + b2_ref[...])
    t = _ssp(t)
    o_ref[...] = jnp.dot(t, w_ref[...], preferred_element_type=jnp.float32) + b_ref[...]


def _tail(a0, a1, lin2_Wa, lin2_Wb, lin2_b, lin_W, lin_b):
    return pl.pallas_call(
        _tail_body,
        grid=(N // BN,),
        in_specs=[
            pl.BlockSpec((BN, HH), lambda i: (i, 0)),
            pl.BlockSpec((BN, HH), lambda i: (i, 0)),
            pl.BlockSpec((HH, H), lambda i: (0, 0)),
            pl.BlockSpec((HH, H), lambda i: (0, 0)),
            pl.BlockSpec((1, H), lambda i: (0, 0)),
            pl.BlockSpec((H, H), lambda i: (0, 0)),
            pl.BlockSpec((1, H), lambda i: (0, 0)),
        ],
        out_specs=pl.BlockSpec((BN, H), lambda i: (i, 0)),
        out_shape=jax.ShapeDtypeStruct((N, H), jnp.float32),
    )(a0, a1, lin2_Wa, lin2_Wb, lin2_b, lin_W, lin_b)


def kernel(x, edge_index, edge_weight, edge_attr, colors,
           mlp_W1, mlp_b1, mlp_W2, mlp_b2,
           lin1_W, lin2_W, lin2_b, lin_W, lin_b):
    assert x.shape == (N, H) and edge_attr.shape == (E, G)
    src = edge_index[0]
    dst = edge_index[1]
    eids = jnp.arange(E, dtype=jnp.int32)
    ea_pad = jnp.pad(edge_attr, ((0, 0), (0, GP - G)))
    w1_pad = jnp.pad(mlp_W1, ((0, 0), (0, GP - G), (0, 0)))
    colors2d = jnp.pad(colors.reshape(NWK, CE), ((0, 0), (0, CEP - CE)),
                       constant_values=NC)

    aux3d, offs3d, mbase = _histbases(colors2d)
    offs = offs3d.reshape(16)
    posE = _ranks(colors2d, mbase)
    posE = jnp.pad(posE.reshape(NWK, CEP), ((0, 0), (0, CEB2 - CEP)),
                   constant_values=EPAD).reshape(NWK * CEB2)
    srcp, dstp, ewp, validp, eidp = _scatter(
        posE, src, dst, edge_weight, eids, aux3d.reshape(NAUX * 16))
    eidp = eidp.astype(jnp.float32).astype(jnp.int32)
    srcp = srcp.astype(jnp.float32).astype(jnp.int32)
    dstp = dstp.astype(jnp.float32).astype(jnp.int32)
    eap = _eagather(ea_pad, eidp)
    w0, wf1 = _mlp(offs, eap, ewp[:EPAD, None], validp[:EPAD, None],
                   w1_pad, mlp_b1[:, None], mlp_W2, mlp_b2[:, None])
    h0, h1 = _lin1(x, lin1_W)
    a0, a1 = _msg_agg(h0, h1, w0, wf1, srcp, dstp)
    out = _tail(a0, a1, lin2_W[:HH], lin2_W[HH:], lin2_b[None],
                lin_W, lin_b[None])
    return out


# speedup vs baseline: 6.3132x; 6.3132x over previous
import functools
from math import pi as PI

import jax
import jax.numpy as jnp
from jax import lax
from jax.experimental import pallas as pl
from jax.experimental.pallas import tpu as pltpu
from jax.experimental.pallas import tpu_sc as plsc

CUTOFF = 10.0
LOG2 = 0.6931471805599453

N = 10000
E = 160000
H = 256
G = 50
GP = 64
FNUM = 256
NC = 4
HH = H // 2

BE = 800
BN = 1000

NUM_TILES = 16
EP = E // NUM_TILES
KCH = 80
NPAD = 10240
NP = NPAD // NUM_TILES
SB = 32


def _ssp(v):
    return jax.nn.softplus(v) - LOG2


def _filters_body(ea_ref, cf_ref, ew_ref, w1_ref, b1_ref, w2_ref, b2_ref,
                  o0_ref, o1_ref):
    ea = ea_ref[...]
    cf = cf_ref[...]
    c_env = 0.5 * (jnp.cos(ew_ref[...] * (PI / CUTOFF)) + 1.0)
    acc = jnp.zeros((BE, FNUM), jnp.float32)
    for c in range(NC):
        h1 = _ssp(jnp.dot(ea, w1_ref[c], preferred_element_type=jnp.float32)
                  + b1_ref[0, c])
        f = jnp.dot(h1, w2_ref[c], preferred_element_type=jnp.float32) + b2_ref[0, c]
        acc = jnp.where(cf == float(c), f, acc)
    acc = acc * c_env
    o0_ref[...] = acc[:, :HH]
    o1_ref[...] = acc[:, HH:]


def _filters(ea_pad, colors_f, ew_col, w1, b1, w2, b2):
    grid = (E // BE,)
    return pl.pallas_call(
        _filters_body,
        grid=grid,
        in_specs=[
            pl.BlockSpec((BE, GP), lambda i: (i, 0)),
            pl.BlockSpec((BE, 1), lambda i: (i, 0)),
            pl.BlockSpec((BE, 1), lambda i: (i, 0)),
            pl.BlockSpec((NC, GP, FNUM), lambda i: (0, 0, 0)),
            pl.BlockSpec((1, NC, FNUM), lambda i: (0, 0, 0)),
            pl.BlockSpec((NC, FNUM, FNUM), lambda i: (0, 0, 0)),
            pl.BlockSpec((1, NC, FNUM), lambda i: (0, 0, 0)),
        ],
        out_specs=[
            pl.BlockSpec((BE, HH), lambda i: (i, 0)),
            pl.BlockSpec((BE, HH), lambda i: (i, 0)),
        ],
        out_shape=[
            jax.ShapeDtypeStruct((E, HH), jnp.float32),
            jax.ShapeDtypeStruct((E, HH), jnp.float32),
        ],
    )(ea_pad, colors_f, ew_col, w1, b1, w2, b2)


def _lin1_body(x_ref, w_ref, o0_ref, o1_ref):
    h = jnp.dot(x_ref[...], w_ref[...], preferred_element_type=jnp.float32)
    o0_ref[...] = h[:, :HH]
    o1_ref[...] = h[:, HH:]


def _lin1(x, lin1_W):
    return pl.pallas_call(
        _lin1_body,
        grid=(N // BN,),
        in_specs=[
            pl.BlockSpec((BN, H), lambda i: (i, 0)),
            pl.BlockSpec((H, FNUM), lambda i: (0, 0)),
        ],
        out_specs=[
            pl.BlockSpec((BN, HH), lambda i: (i, 0)),
            pl.BlockSpec((BN, HH), lambda i: (i, 0)),
        ],
        out_shape=[
            jax.ShapeDtypeStruct((N, HH), jnp.float32),
            jax.ShapeDtypeStruct((N, HH), jnp.float32),
        ],
    )(x, lin1_W)


def _zero_rows(buf, rows):
    z = jnp.zeros((16,), jnp.float32)

    def zrow(r, carry):
        for c8 in range(HH // 16):
            buf[r, pl.ds(c8 * 16, 16)] = z
        return carry

    lax.fori_loop(0, rows, zrow, 0)


def _msg_half(h_hbm, w_hbm, src_hbm, dst_hbm, agg_hbm,
              srcbufs, dstbufs, hrowss, wrowss, sbuf, acc, semh, semw, sid):
    base = sid * EP
    nch = EP // KCH

    def issue(k, b):
        off = base + k * KCH
        pltpu.sync_copy(src_hbm.at[pl.ds(off, KCH)], srcbufs[b])
        pltpu.sync_copy(dst_hbm.at[pl.ds(off, KCH)], dstbufs[b])
        pltpu.async_copy(h_hbm.at[srcbufs[b]], hrowss[b], semh[b])
        pltpu.async_copy(w_hbm.at[pl.ds(off, KCH)], wrowss[b], semw[b])

    def process(b):
        pltpu.make_async_copy(h_hbm.at[srcbufs[b]], hrowss[b], semh[b]).wait()
        pltpu.make_async_copy(w_hbm.at[pl.ds(0, KCH)], wrowss[b], semw[b]).wait()

        def mrow(r, c2):
            for c8 in range(HH // 16):
                sl = pl.ds(c8 * 16, 16)
                wrowss[b][r, sl] = wrowss[b][r, sl] * hrowss[b][r, sl]
            return c2

        lax.fori_loop(0, KCH, mrow, 0)
        pltpu.sync_copy(wrowss[b], acc.at[dstbufs[b]], add=True)

    issue(0, 0)
    issue(1, 1)

    def pair(k2, carry):
        k = k2 * 2
        for b in range(2):
            process(b)

            @pl.when(k + 2 + b < nch)
            def _(b=b):
                issue(k + 2 + b, b)
        return carry

    lax.fori_loop(0, nch // 2, pair, 0)
    if nch % 2 == 1:
        process(0)
    plsc.subcore_barrier()
    for r in range(NP // SB):
        sl = pl.ds(sid * NP + r * SB, SB)
        cp = pltpu.async_copy(acc.at[sl], sbuf, semh[0])
        cp.wait()
        pltpu.sync_copy(sbuf, agg_hbm.at[sl])


def _msg_kernel_body(h0, h1, w0, w1, src_hbm, dst_hbm, agg0, agg1,
                     srcbuf0, srcbuf1, dstbuf0, dstbuf1,
                     hrows0, hrows1, wrows0, wrows1, sbuf, acc,
                     semh0, semh1, semw0, semw1):
    cid = lax.axis_index("c")
    sid = lax.axis_index("s")
    z = jnp.zeros((16,), jnp.float32)

    def zrow(r, carry):
        for c8 in range(HH // 16):
            sbuf[r, pl.ds(c8 * 16, 16)] = z
        return carry

    lax.fori_loop(0, SB, zrow, 0)
    for r in range(NP // SB):
        pltpu.sync_copy(sbuf, acc.at[pl.ds(sid * NP + r * SB, SB)])
    plsc.subcore_barrier()

    srcbufs = (srcbuf0, srcbuf1)
    dstbufs = (dstbuf0, dstbuf1)
    hrowss = (hrows0, hrows1)
    wrowss = (wrows0, wrows1)
    semh = (semh0, semh1)
    semw = (semw0, semw1)

    @pl.when(cid == 0)
    def _():
        _msg_half(h0, w0, src_hbm, dst_hbm, agg0,
                  srcbufs, dstbufs, hrowss, wrowss, sbuf, acc, semh, semw, sid)

    @pl.when(cid == 1)
    def _():
        _msg_half(h1, w1, src_hbm, dst_hbm, agg1,
                  srcbufs, dstbufs, hrowss, wrowss, sbuf, acc, semh, semw, sid)


def _msg_agg(h0, h1, w0, w1, src, dst):
    mesh = plsc.VectorSubcoreMesh(core_axis_name="c", subcore_axis_name="s", num_cores=2, num_subcores=16)
    f = pl.kernel(
        _msg_kernel_body,
        out_type=[
            jax.ShapeDtypeStruct((NPAD, HH), jnp.float32),
            jax.ShapeDtypeStruct((NPAD, HH), jnp.float32),
        ],
        mesh=mesh,
        scratch_types=[
            pltpu.VMEM((KCH,), jnp.int32),
            pltpu.VMEM((KCH,), jnp.int32),
            pltpu.VMEM((KCH,), jnp.int32),
            pltpu.VMEM((KCH,), jnp.int32),
            pltpu.VMEM((KCH, HH), jnp.float32),
            pltpu.VMEM((KCH, HH), jnp.float32),
            pltpu.VMEM((KCH, HH), jnp.float32),
            pltpu.VMEM((KCH, HH), jnp.float32),
            pltpu.VMEM((SB, HH), jnp.float32),
            pltpu.VMEM_SHARED((NPAD, HH), jnp.float32),
            pltpu.SemaphoreType.DMA,
            pltpu.SemaphoreType.DMA,
            pltpu.SemaphoreType.DMA,
            pltpu.SemaphoreType.DMA,
        ],
    )
    return f(h0, h1, w0, w1, src, dst)


def _tail_body(a0_ref, a1_ref, w2a_ref, w2b_ref, b2_ref, w_ref, b_ref, o_ref):
    t = (jnp.dot(a0_ref[...], w2a_ref[...], preferred_element_type=jnp.float32)
         + jnp.dot(a1_ref[...], w2b_ref[...], preferred_element_type=jnp.float32)
         + b2_ref[...])
    t = _ssp(t)
    o_ref[...] = jnp.dot(t, w_ref[...], preferred_element_type=jnp.float32) + b_ref[...]


def _tail(a0, a1, lin2_Wa, lin2_Wb, lin2_b, lin_W, lin_b):
    return pl.pallas_call(
        _tail_body,
        grid=(N // BN,),
        in_specs=[
            pl.BlockSpec((BN, HH), lambda i: (i, 0)),
            pl.BlockSpec((BN, HH), lambda i: (i, 0)),
            pl.BlockSpec((HH, H), lambda i: (0, 0)),
            pl.BlockSpec((HH, H), lambda i: (0, 0)),
            pl.BlockSpec((1, H), lambda i: (0, 0)),
            pl.BlockSpec((H, H), lambda i: (0, 0)),
            pl.BlockSpec((1, H), lambda i: (0, 0)),
        ],
        out_specs=pl.BlockSpec((BN, H), lambda i: (i, 0)),
        out_shape=jax.ShapeDtypeStruct((N, H), jnp.float32),
    )(a0, a1, lin2_Wa, lin2_Wb, lin2_b, lin_W, lin_b)


def kernel(x, edge_index, edge_weight, edge_attr, colors,
           mlp_W1, mlp_b1, mlp_W2, mlp_b2,
           lin1_W, lin2_W, lin2_b, lin_W, lin_b):
    assert x.shape == (N, H) and edge_attr.shape == (E, G)
    src = edge_index[0]
    dst = edge_index[1]
    ea_pad = jnp.pad(edge_attr, ((0, 0), (0, GP - G)))
    w1_pad = jnp.pad(mlp_W1, ((0, 0), (0, GP - G), (0, 0)))
    colors_f = colors.astype(jnp.float32)[:, None]
    ew_col = edge_weight[:, None]

    w0, wf1 = _filters(ea_pad, colors_f, ew_col,
                       w1_pad, mlp_b1[None], mlp_W2, mlp_b2[None])
    h0, h1 = _lin1(x, lin1_W)
    a0, a1 = _msg_agg(h0, h1, w0, wf1, src, dst)
    out = _tail(a0, a1, lin2_W[:HH], lin2_W[HH:], lin2_b[None],
                lin_W, lin_b[None])
    return out


# speedup vs baseline: 6.4526x; 1.0221x over previous
import functools
from math import pi as PI

import jax
import jax.numpy as jnp
from jax import lax
from jax.experimental import pallas as pl
from jax.experimental.pallas import tpu as pltpu
from jax.experimental.pallas import tpu_sc as plsc

CUTOFF = 10.0
LOG2 = 0.6931471805599453

N = 10000
E = 160000
H = 256
G = 50
GP = 64
FNUM = 256
NC = 4
HH = H // 2

BE = 1600
BN = 1000

NUM_TILES = 16
EP = E // NUM_TILES
KCH = 80
NPAD = 10240
NP = NPAD // NUM_TILES
SB = 32


def _ssp(v):
    return jax.nn.softplus(v) - LOG2


def _filters_body(ea_ref, cf_ref, ew_ref, w1_ref, b1_ref, w2_ref, b2_ref,
                  o0_ref, o1_ref):
    ea = ea_ref[...]
    cf = cf_ref[...]
    c_env = 0.5 * (jnp.cos(ew_ref[...] * (PI / CUTOFF)) + 1.0)
    acc = jnp.zeros((BE, FNUM), jnp.float32)
    for c in range(NC):
        h1 = _ssp(jnp.dot(ea, w1_ref[c], preferred_element_type=jnp.float32)
                  + b1_ref[0, c])
        f = jnp.dot(h1, w2_ref[c], preferred_element_type=jnp.float32) + b2_ref[0, c]
        acc = jnp.where(cf == float(c), f, acc)
    acc = acc * c_env
    o0_ref[...] = acc[:, :HH]
    o1_ref[...] = acc[:, HH:]


def _filters(ea_pad, colors_f, ew_col, w1, b1, w2, b2):
    grid = (E // BE,)
    return pl.pallas_call(
        _filters_body,
        grid=grid,
        in_specs=[
            pl.BlockSpec((BE, GP), lambda i: (i, 0)),
            pl.BlockSpec((BE, 1), lambda i: (i, 0)),
            pl.BlockSpec((BE, 1), lambda i: (i, 0)),
            pl.BlockSpec((NC, GP, FNUM), lambda i: (0, 0, 0)),
            pl.BlockSpec((1, NC, FNUM), lambda i: (0, 0, 0)),
            pl.BlockSpec((NC, FNUM, FNUM), lambda i: (0, 0, 0)),
            pl.BlockSpec((1, NC, FNUM), lambda i: (0, 0, 0)),
        ],
        out_specs=[
            pl.BlockSpec((BE, HH), lambda i: (i, 0)),
            pl.BlockSpec((BE, HH), lambda i: (i, 0)),
        ],
        out_shape=[
            jax.ShapeDtypeStruct((E, HH), jnp.float32),
            jax.ShapeDtypeStruct((E, HH), jnp.float32),
        ],
    )(ea_pad, colors_f, ew_col, w1, b1, w2, b2)


def _lin1_body(x_ref, w_ref, o0_ref, o1_ref):
    h = jnp.dot(x_ref[...], w_ref[...], preferred_element_type=jnp.float32)
    o0_ref[...] = h[:, :HH]
    o1_ref[...] = h[:, HH:]


def _lin1(x, lin1_W):
    return pl.pallas_call(
        _lin1_body,
        grid=(N // BN,),
        in_specs=[
            pl.BlockSpec((BN, H), lambda i: (i, 0)),
            pl.BlockSpec((H, FNUM), lambda i: (0, 0)),
        ],
        out_specs=[
            pl.BlockSpec((BN, HH), lambda i: (i, 0)),
            pl.BlockSpec((BN, HH), lambda i: (i, 0)),
        ],
        out_shape=[
            jax.ShapeDtypeStruct((N, HH), jnp.float32),
            jax.ShapeDtypeStruct((N, HH), jnp.float32),
        ],
    )(x, lin1_W)


def _zero_rows(buf, rows):
    z = jnp.zeros((16,), jnp.float32)

    def zrow(r, carry):
        for c8 in range(HH // 16):
            buf[r, pl.ds(c8 * 16, 16)] = z
        return carry

    lax.fori_loop(0, rows, zrow, 0)


def _msg_half(h_hbm, w_hbm, src_hbm, dst_hbm, agg_hbm,
              srcbufs, dstbufs, hrowss, wrowss, sbuf, acc, semh, semw, sid):
    base = sid * EP
    nch = EP // KCH

    def issue(k, b):
        off = base + k * KCH
        pltpu.sync_copy(src_hbm.at[pl.ds(off, KCH)], srcbufs[b])
        pltpu.sync_copy(dst_hbm.at[pl.ds(off, KCH)], dstbufs[b])
        pltpu.async_copy(h_hbm.at[srcbufs[b]], hrowss[b], semh[b])
        pltpu.async_copy(w_hbm.at[pl.ds(off, KCH)], wrowss[b], semw[b])

    def process(b):
        pltpu.make_async_copy(h_hbm.at[srcbufs[b]], hrowss[b], semh[b]).wait()
        pltpu.make_async_copy(w_hbm.at[pl.ds(0, KCH)], wrowss[b], semw[b]).wait()

        def mrow(r, c2):
            for c8 in range(HH // 16):
                sl = pl.ds(c8 * 16, 16)
                wrowss[b][r, sl] = wrowss[b][r, sl] * hrowss[b][r, sl]
            return c2

        lax.fori_loop(0, KCH, mrow, 0)
        pltpu.sync_copy(wrowss[b], acc.at[dstbufs[b]], add=True)

    issue(0, 0)
    issue(1, 1)

    def pair(k2, carry):
        k = k2 * 2
        for b in range(2):
            process(b)

            @pl.when(k + 2 + b < nch)
            def _(b=b):
                issue(k + 2 + b, b)
        return carry

    lax.fori_loop(0, nch // 2, pair, 0)
    if nch % 2 == 1:
        process(0)
    plsc.subcore_barrier()
    for r in range(NP // SB):
        sl = pl.ds(sid * NP + r * SB, SB)
        cp = pltpu.async_copy(acc.at[sl], sbuf, semh[0])
        cp.wait()
        pltpu.sync_copy(sbuf, agg_hbm.at[sl])


def _msg_kernel_body(h0, h1, w0, w1, src_hbm, dst_hbm, agg0, agg1,
                     srcbuf0, srcbuf1, dstbuf0, dstbuf1,
                     hrows0, hrows1, wrows0, wrows1, sbuf, acc,
                     semh0, semh1, semw0, semw1):
    cid = lax.axis_index("c")
    sid = lax.axis_index("s")
    z = jnp.zeros((16,), jnp.float32)

    def zrow(r, carry):
        for c8 in range(HH // 16):
            sbuf[r, pl.ds(c8 * 16, 16)] = z
        return carry

    lax.fori_loop(0, SB, zrow, 0)
    for r in range(NP // SB):
        pltpu.sync_copy(sbuf, acc.at[pl.ds(sid * NP + r * SB, SB)])
    plsc.subcore_barrier()

    srcbufs = (srcbuf0, srcbuf1)
    dstbufs = (dstbuf0, dstbuf1)
    hrowss = (hrows0, hrows1)
    wrowss = (wrows0, wrows1)
    semh = (semh0, semh1)
    semw = (semw0, semw1)

    @pl.when(cid == 0)
    def _():
        _msg_half(h0, w0, src_hbm, dst_hbm, agg0,
                  srcbufs, dstbufs, hrowss, wrowss, sbuf, acc, semh, semw, sid)

    @pl.when(cid == 1)
    def _():
        _msg_half(h1, w1, src_hbm, dst_hbm, agg1,
                  srcbufs, dstbufs, hrowss, wrowss, sbuf, acc, semh, semw, sid)


def _msg_agg(h0, h1, w0, w1, src, dst):
    mesh = plsc.VectorSubcoreMesh(core_axis_name="c", subcore_axis_name="s", num_cores=2, num_subcores=16)
    f = pl.kernel(
        _msg_kernel_body,
        out_type=[
            jax.ShapeDtypeStruct((NPAD, HH), jnp.float32),
            jax.ShapeDtypeStruct((NPAD, HH), jnp.float32),
        ],
        mesh=mesh,
        scratch_types=[
            pltpu.VMEM((KCH,), jnp.int32),
            pltpu.VMEM((KCH,), jnp.int32),
            pltpu.VMEM((KCH,), jnp.int32),
            pltpu.VMEM((KCH,), jnp.int32),
            pltpu.VMEM((KCH, HH), jnp.float32),
            pltpu.VMEM((KCH, HH), jnp.float32),
            pltpu.VMEM((KCH, HH), jnp.float32),
            pltpu.VMEM((KCH, HH), jnp.float32),
            pltpu.VMEM((SB, HH), jnp.float32),
            pltpu.VMEM_SHARED((NPAD, HH), jnp.float32),
            pltpu.SemaphoreType.DMA,
            pltpu.SemaphoreType.DMA,
            pltpu.SemaphoreType.DMA,
            pltpu.SemaphoreType.DMA,
        ],
    )
    return f(h0, h1, w0, w1, src, dst)


def _tail_body(a0_ref, a1_ref, w2a_ref, w2b_ref, b2_ref, w_ref, b_ref, o_ref):
    t = (jnp.dot(a0_ref[...], w2a_ref[...], preferred_element_type=jnp.float32)
         + jnp.dot(a1_ref[...], w2b_ref[...], preferred_element_type=jnp.float32)
         + b2_ref[...])
    t = _ssp(t)
    o_ref[...] = jnp.dot(t, w_ref[...], preferred_element_type=jnp.float32) + b_ref[...]


def _tail(a0, a1, lin2_Wa, lin2_Wb, lin2_b, lin_W, lin_b):
    return pl.pallas_call(
        _tail_body,
        grid=(N // BN,),
        in_specs=[
            pl.BlockSpec((BN, HH), lambda i: (i, 0)),
            pl.BlockSpec((BN, HH), lambda i: (i, 0)),
            pl.BlockSpec((HH, H), lambda i: (0, 0)),
            pl.BlockSpec((HH, H), lambda i: (0, 0)),
            pl.BlockSpec((1, H), lambda i: (0, 0)),
            pl.BlockSpec((H, H), lambda i: (0, 0)),
            pl.BlockSpec((1, H), lambda i: (0, 0)),
        ],
        out_specs=pl.BlockSpec((BN, H), lambda i: (i, 0)),
        out_shape=jax.ShapeDtypeStruct((N, H), jnp.float32),
    )(a0, a1, lin2_Wa, lin2_Wb, lin2_b, lin_W, lin_b)


def kernel(x, edge_index, edge_weight, edge_attr, colors,
           mlp_W1, mlp_b1, mlp_W2, mlp_b2,
           lin1_W, lin2_W, lin2_b, lin_W, lin_b):
    assert x.shape == (N, H) and edge_attr.shape == (E, G)
    src = edge_index[0]
    dst = edge_index[1]
    ea_pad = jnp.pad(edge_attr, ((0, 0), (0, GP - G)))
    w1_pad = jnp.pad(mlp_W1, ((0, 0), (0, GP - G), (0, 0)))
    colors_f = colors.astype(jnp.float32)[:, None]
    ew_col = edge_weight[:, None]

    w0, wf1 = _filters(ea_pad, colors_f, ew_col,
                       w1_pad, mlp_b1[None], mlp_W2, mlp_b2[None])
    h0, h1 = _lin1(x, lin1_W)
    a0, a1 = _msg_agg(h0, h1, w0, wf1, src, dst)
    out = _tail(a0, a1, lin2_W[:HH], lin2_W[HH:], lin2_b[None],
                lin_W, lin_b[None])
    return out


# speedup vs baseline: 6.5548x; 1.0158x over previous
import functools
from math import pi as PI

import jax
import jax.numpy as jnp
from jax import lax
from jax.experimental import pallas as pl
from jax.experimental.pallas import tpu as pltpu
from jax.experimental.pallas import tpu_sc as plsc

CUTOFF = 10.0
LOG2 = 0.6931471805599453

N = 10000
E = 160000
H = 256
G = 50
GP = 64
FNUM = 256
NC = 4
HH = H // 2

BE = 3200
BN = 2000

NUM_TILES = 16
EP = E // NUM_TILES
KCH = 80
NPAD = 10240
NP = NPAD // NUM_TILES
SB = 32


def _ssp(v):
    return jax.nn.softplus(v) - LOG2


def _filters_body(ea_ref, cf_ref, ew_ref, w1_ref, b1_ref, w2_ref, b2_ref,
                  o0_ref, o1_ref):
    ea = ea_ref[...]
    cf = cf_ref[...]
    c_env = 0.5 * (jnp.cos(ew_ref[...] * (PI / CUTOFF)) + 1.0)
    acc = jnp.zeros((BE, FNUM), jnp.float32)
    for c in range(NC):
        h1 = _ssp(jnp.dot(ea, w1_ref[c], preferred_element_type=jnp.float32)
                  + b1_ref[0, c])
        f = jnp.dot(h1, w2_ref[c], preferred_element_type=jnp.float32) + b2_ref[0, c]
        acc = jnp.where(cf == float(c), f, acc)
    acc = acc * c_env
    o0_ref[...] = acc[:, :HH]
    o1_ref[...] = acc[:, HH:]


def _filters(ea_pad, colors_f, ew_col, w1, b1, w2, b2):
    grid = (E // BE,)
    return pl.pallas_call(
        _filters_body,
        grid=grid,
        in_specs=[
            pl.BlockSpec((BE, GP), lambda i: (i, 0)),
            pl.BlockSpec((BE, 1), lambda i: (i, 0)),
            pl.BlockSpec((BE, 1), lambda i: (i, 0)),
            pl.BlockSpec((NC, GP, FNUM), lambda i: (0, 0, 0)),
            pl.BlockSpec((1, NC, FNUM), lambda i: (0, 0, 0)),
            pl.BlockSpec((NC, FNUM, FNUM), lambda i: (0, 0, 0)),
            pl.BlockSpec((1, NC, FNUM), lambda i: (0, 0, 0)),
        ],
        out_specs=[
            pl.BlockSpec((BE, HH), lambda i: (i, 0)),
            pl.BlockSpec((BE, HH), lambda i: (i, 0)),
        ],
        out_shape=[
            jax.ShapeDtypeStruct((E, HH), jnp.float32),
            jax.ShapeDtypeStruct((E, HH), jnp.float32),
        ],
    )(ea_pad, colors_f, ew_col, w1, b1, w2, b2)


def _lin1_body(x_ref, w_ref, o0_ref, o1_ref):
    h = jnp.dot(x_ref[...], w_ref[...], preferred_element_type=jnp.float32)
    o0_ref[...] = h[:, :HH]
    o1_ref[...] = h[:, HH:]


def _lin1(x, lin1_W):
    return pl.pallas_call(
        _lin1_body,
        grid=(N // BN,),
        in_specs=[
            pl.BlockSpec((BN, H), lambda i: (i, 0)),
            pl.BlockSpec((H, FNUM), lambda i: (0, 0)),
        ],
        out_specs=[
            pl.BlockSpec((BN, HH), lambda i: (i, 0)),
            pl.BlockSpec((BN, HH), lambda i: (i, 0)),
        ],
        out_shape=[
            jax.ShapeDtypeStruct((N, HH), jnp.float32),
            jax.ShapeDtypeStruct((N, HH), jnp.float32),
        ],
    )(x, lin1_W)


def _zero_rows(buf, rows):
    z = jnp.zeros((16,), jnp.float32)

    def zrow(r, carry):
        for c8 in range(HH // 16):
            buf[r, pl.ds(c8 * 16, 16)] = z
        return carry

    lax.fori_loop(0, rows, zrow, 0)


def _msg_half(h_hbm, w_hbm, src_hbm, dst_hbm, agg_hbm,
              srcbufs, dstbufs, hrowss, wrowss, sbuf, acc, semh, semw, sid):
    base = sid * EP
    nch = EP // KCH

    def issue(k, b):
        off = base + k * KCH
        pltpu.sync_copy(src_hbm.at[pl.ds(off, KCH)], srcbufs[b])
        pltpu.sync_copy(dst_hbm.at[pl.ds(off, KCH)], dstbufs[b])
        pltpu.async_copy(h_hbm.at[srcbufs[b]], hrowss[b], semh[b])
        pltpu.async_copy(w_hbm.at[pl.ds(off, KCH)], wrowss[b], semw[b])

    def process(b):
        pltpu.make_async_copy(h_hbm.at[srcbufs[b]], hrowss[b], semh[b]).wait()
        pltpu.make_async_copy(w_hbm.at[pl.ds(0, KCH)], wrowss[b], semw[b]).wait()

        def mrow(r, c2):
            for c8 in range(HH // 16):
                sl = pl.ds(c8 * 16, 16)
                wrowss[b][r, sl] = wrowss[b][r, sl] * hrowss[b][r, sl]
            return c2

        lax.fori_loop(0, KCH, mrow, 0)
        pltpu.sync_copy(wrowss[b], acc.at[dstbufs[b]], add=True)

    issue(0, 0)
    issue(1, 1)

    def pair(k2, carry):
        k = k2 * 2
        for b in range(2):
            process(b)

            @pl.when(k + 2 + b < nch)
            def _(b=b):
                issue(k + 2 + b, b)
        return carry

    lax.fori_loop(0, nch // 2, pair, 0)
    if nch % 2 == 1:
        process(0)
    plsc.subcore_barrier()
    for r in range(NP // SB):
        sl = pl.ds(sid * NP + r * SB, SB)
        cp = pltpu.async_copy(acc.at[sl], sbuf, semh[0])
        cp.wait()
        pltpu.sync_copy(sbuf, agg_hbm.at[sl])


def _msg_kernel_body(h0, h1, w0, w1, src_hbm, dst_hbm, agg0, agg1,
                     srcbuf0, srcbuf1, dstbuf0, dstbuf1,
                     hrows0, hrows1, wrows0, wrows1, sbuf, acc,
                     semh0, semh1, semw0, semw1):
    cid = lax.axis_index("c")
    sid = lax.axis_index("s")
    z = jnp.zeros((16,), jnp.float32)

    def zrow(r, carry):
        for c8 in range(HH // 16):
            sbuf[r, pl.ds(c8 * 16, 16)] = z
        return carry

    lax.fori_loop(0, SB, zrow, 0)
    for r in range(NP // SB):
        pltpu.sync_copy(sbuf, acc.at[pl.ds(sid * NP + r * SB, SB)])
    plsc.subcore_barrier()

    srcbufs = (srcbuf0, srcbuf1)
    dstbufs = (dstbuf0, dstbuf1)
    hrowss = (hrows0, hrows1)
    wrowss = (wrows0, wrows1)
    semh = (semh0, semh1)
    semw = (semw0, semw1)

    @pl.when(cid == 0)
    def _():
        _msg_half(h0, w0, src_hbm, dst_hbm, agg0,
                  srcbufs, dstbufs, hrowss, wrowss, sbuf, acc, semh, semw, sid)

    @pl.when(cid == 1)
    def _():
        _msg_half(h1, w1, src_hbm, dst_hbm, agg1,
                  srcbufs, dstbufs, hrowss, wrowss, sbuf, acc, semh, semw, sid)


def _msg_agg(h0, h1, w0, w1, src, dst):
    mesh = plsc.VectorSubcoreMesh(core_axis_name="c", subcore_axis_name="s", num_cores=2, num_subcores=16)
    f = pl.kernel(
        _msg_kernel_body,
        out_type=[
            jax.ShapeDtypeStruct((NPAD, HH), jnp.float32),
            jax.ShapeDtypeStruct((NPAD, HH), jnp.float32),
        ],
        mesh=mesh,
        scratch_types=[
            pltpu.VMEM((KCH,), jnp.int32),
            pltpu.VMEM((KCH,), jnp.int32),
            pltpu.VMEM((KCH,), jnp.int32),
            pltpu.VMEM((KCH,), jnp.int32),
            pltpu.VMEM((KCH, HH), jnp.float32),
            pltpu.VMEM((KCH, HH), jnp.float32),
            pltpu.VMEM((KCH, HH), jnp.float32),
            pltpu.VMEM((KCH, HH), jnp.float32),
            pltpu.VMEM((SB, HH), jnp.float32),
            pltpu.VMEM_SHARED((NPAD, HH), jnp.float32),
            pltpu.SemaphoreType.DMA,
            pltpu.SemaphoreType.DMA,
            pltpu.SemaphoreType.DMA,
            pltpu.SemaphoreType.DMA,
        ],
    )
    return f(h0, h1, w0, w1, src, dst)


def _tail_body(a0_ref, a1_ref, w2a_ref, w2b_ref, b2_ref, w_ref, b_ref, o_ref):
    t = (jnp.dot(a0_ref[...], w2a_ref[...], preferred_element_type=jnp.float32)
         + jnp.dot(a1_ref[...], w2b_ref[...], preferred_element_type=jnp.float32)
         + b2_ref[...])
    t = _ssp(t)
    o_ref[...] = jnp.dot(t, w_ref[...], preferred_element_type=jnp.float32) + b_ref[...]


def _tail(a0, a1, lin2_Wa, lin2_Wb, lin2_b, lin_W, lin_b):
    return pl.pallas_call(
        _tail_body,
        grid=(N // BN,),
        in_specs=[
            pl.BlockSpec((BN, HH), lambda i: (i, 0)),
            pl.BlockSpec((BN, HH), lambda i: (i, 0)),
            pl.BlockSpec((HH, H), lambda i: (0, 0)),
            pl.BlockSpec((HH, H), lambda i: (0, 0)),
            pl.BlockSpec((1, H), lambda i: (0, 0)),
            pl.BlockSpec((H, H), lambda i: (0, 0)),
            pl.BlockSpec((1, H), lambda i: (0, 0)),
        ],
        out_specs=pl.BlockSpec((BN, H), lambda i: (i, 0)),
        out_shape=jax.ShapeDtypeStruct((N, H), jnp.float32),
    )(a0, a1, lin2_Wa, lin2_Wb, lin2_b, lin_W, lin_b)


def kernel(x, edge_index, edge_weight, edge_attr, colors,
           mlp_W1, mlp_b1, mlp_W2, mlp_b2,
           lin1_W, lin2_W, lin2_b, lin_W, lin_b):
    assert x.shape == (N, H) and edge_attr.shape == (E, G)
    src = edge_index[0]
    dst = edge_index[1]
    ea_pad = jnp.pad(edge_attr, ((0, 0), (0, GP - G)))
    w1_pad = jnp.pad(mlp_W1, ((0, 0), (0, GP - G), (0, 0)))
    colors_f = colors.astype(jnp.float32)[:, None]
    ew_col = edge_weight[:, None]

    w0, wf1 = _filters(ea_pad, colors_f, ew_col,
                       w1_pad, mlp_b1[None], mlp_W2, mlp_b2[None])
    h0, h1 = _lin1(x, lin1_W)
    a0, a1 = _msg_agg(h0, h1, w0, wf1, src, dst)
    out = _tail(a0, a1, lin2_W[:HH], lin2_W[HH:], lin2_b[None],
                lin_W, lin_b[None])
    return out
